# baseline (speedup 1.0000x reference)
import functools

import jax
import jax.numpy as jnp
from jax import lax
from jax.experimental import pallas as pl
from jax.experimental.pallas import tpu as pltpu
from jax.experimental.pallas import tpu_sc as plsc

SEEDS_P1 = (1609.0, 5154.0, 6527.0, 2426.0)
HASH_RANGE = 65536
NUM_HEADS = 4
EMBED_DIM = 64
LANES = 16
NC, NS = 2, 16
NW = NC * NS
N_SUB = 8
SUB = HASH_RANGE // N_SUB

B, T = 4, 512
TPAD = T + 8
POS = B * T
NA = B * (T - 1)
NB = B * (T - 2)
NAV = (NA + LANES - 1) // LANES
NBV = (NB + LANES - 1) // LANES
LIST = NAV * LANES + NBV * LANES
SENTINEL = HASH_RANGE


def _gather_body(tok_hbm, table_hbm, out_hbm,
                 tok_v, idxl, posl, buf, acc):
    wid = lax.axis_index("s") * NC + lax.axis_index("c")
    h = wid // N_SUB
    ca = wid - h * N_SUB
    seed = jnp.float32(0.0)
    for i in range(NUM_HEADS):
        seed = jnp.where(h == i, jnp.float32(SEEDS_P1[i]), seed)

    pltpu.sync_copy(tok_hbm, tok_v)
    iota = lax.iota(jnp.int32, LANES)
    zeros = jnp.zeros((LANES,), jnp.float32)

    def zbody(j, c):
        for ci in range(N_SUB):
            acc[ci, pl.ds(j * LANES, LANES)] = zeros
        return c
    lax.fori_loop(0, POS // LANES, zbody, 0)

    def hash_stream(base_slot, n_entries, win_len, is_tri):
        magic = (1 << 21) // win_len + 1

        def body(i, c):
            e = i * LANES + iota
            valid = e < n_entries
            e = jnp.minimum(e, n_entries - 1)
            b = (e * magic) >> 21
            w = e - b * win_len
            p = b * T + w
            ti = b * TPAD + w
            t0 = plsc.load_gather(tok_v, [ti]).astype(jnp.float32)
            t1 = plsc.load_gather(tok_v, [ti + 1]).astype(jnp.float32)
            hv = t0 * seed + t1 * seed
            if is_tri:
                t2 = plsc.load_gather(tok_v, [ti + 2]).astype(jnp.float32)
                hv = hv + t2 * seed
            hidx = hv.astype(jnp.int32) & (HASH_RANGE - 1)
            hidx = jnp.where(valid, hidx, SENTINEL)
            idxl[pl.ds(base_slot + i * LANES, LANES)] = hidx
            posl[pl.ds(base_slot + i * LANES, LANES)] = p
            return c
        return body

    lax.fori_loop(0, NAV, hash_stream(0, NA, T - 1, False), 0)
    lax.fori_loop(0, NBV, hash_stream(NAV * LANES, NB, T - 2, True), 0)

    nlv = LIST // LANES
    ci_vecs = [iota * 0 + ci for ci in range(N_SUB)]

    for s in range(N_SUB):
        pltpu.sync_copy(
            table_hbm.at[h, pl.ds(ca * N_SUB, N_SUB), pl.ds(s * SUB, SUB)],
            buf)

        def gbody(i, c):
            lv = idxl[pl.ds(i * LANES, LANES)]
            pv = posl[pl.ds(i * LANES, LANES)]
            m = (lv >> 13) == s
            rl = lv & (SUB - 1)
            for ci in range(N_SUB):
                val = plsc.load_gather(buf, [ci_vecs[ci], rl], mask=m)
                plsc.addupdate_scatter(acc, [ci_vecs[ci], pv], val, mask=m)
            return c
        lax.fori_loop(0, nlv, gbody, 0)

    pltpu.sync_copy(acc, out_hbm.at[pl.ds(wid * N_SUB, N_SUB)])


def _make_gather():
    mesh = plsc.VectorSubcoreMesh(
        core_axis_name="c", subcore_axis_name="s",
        num_cores=NC, num_subcores=NS)
    return pl.kernel(
        _gather_body,
        out_type=jax.ShapeDtypeStruct((NW * N_SUB, POS), jnp.float32),
        mesh=mesh,
        scratch_types=[
            pltpu.VMEM((B * TPAD,), jnp.int32),
            pltpu.VMEM((LIST,), jnp.int32),
            pltpu.VMEM((LIST,), jnp.int32),
            pltpu.VMEM((N_SUB, SUB), jnp.float32),
            pltpu.VMEM((N_SUB, POS), jnp.float32),
        ],
        compiler_params=pltpu.CompilerParams(
            use_tc_tiling_on_sc=True, needs_layout_passes=False),
    )


def _dense_body(parts_ref, hid_ref, wh_ref, bh_ref, wg1_ref, bg1_ref,
                wg2_ref, bg2_ref, out_ref):
    parts = parts_ref[...]
    seq_t = parts[0:64] + parts[64:128] + parts[128:192] + parts[192:256]
    seq = jnp.transpose(seq_t) * 0.25
    hid = hid_ref[...]
    dn = (((1,), (1,)), ((), ()))
    proj = lax.dot_general(seq, wh_ref[...], dn,
                           preferred_element_type=jnp.float32) + bh_ref[...]
    hmid = hid + proj
    g1 = lax.dot_general(hmid, wg1_ref[...], dn,
                         preferred_element_type=jnp.float32) + bg1_ref[...]
    g1 = 0.5 * g1 * (1.0 + lax.erf(g1 * jnp.float32(0.7071067811865476)))
    g2 = jnp.sum(g1 * wg2_ref[...], axis=1, keepdims=True) + bg2_ref[...]
    gate = jax.nn.sigmoid(g2)
    out_ref[...] = hid + gate * proj


def kernel(token_ids, hidden_state, embeddings, W_hid, b_hid,
           W_g1, b_g1, W_g2, b_g2):
    D = hidden_state.shape[-1]
    tok_pad = jnp.pad(token_ids, ((0, 0), (0, TPAD - T))).reshape(-1)
    table_t = embeddings.transpose(0, 2, 1)
    parts = _make_gather()(tok_pad, table_t)

    out = pl.pallas_call(
        _dense_body,
        out_shape=jax.ShapeDtypeStruct((POS, D), jnp.float32),
    )(
        parts,
        hidden_state.reshape(POS, D),
        W_hid,
        b_hid.reshape(1, D),
        W_g1,
        b_g1.reshape(1, -1),
        W_g2,
        b_g2.reshape(1, 1),
    )
    return out.reshape(B, T, D)


# speedup vs baseline: 1.0286x; 1.0286x over previous
import functools

import jax
import jax.numpy as jnp
from jax import lax
from jax.experimental import pallas as pl
from jax.experimental.pallas import tpu as pltpu
from jax.experimental.pallas import tpu_sc as plsc

SEEDS_P1 = (1609.0, 5154.0, 6527.0, 2426.0)
HASH_RANGE = 65536
NUM_HEADS = 4
EMBED_DIM = 64
LANES = 16
NC, NS = 2, 16
NW = NC * NS
N_SUB = 8
SUB = HASH_RANGE // N_SUB

B, T = 4, 512
TPAD = T + 8
POS = B * T
NA = B * (T - 1)
NB = B * (T - 2)
NAV = (NA + LANES - 1) // LANES
NBV = (NB + LANES - 1) // LANES
LIST = NAV * LANES + NBV * LANES
SENTINEL = HASH_RANGE


def _gather_body(tok_hbm, table_hbm, out_hbm,
                 tok_v, idxl, posl, buf, acc):
    wid = lax.axis_index("s") * NC + lax.axis_index("c")
    h = wid // N_SUB
    ca = wid - h * N_SUB
    seed = jnp.float32(0.0)
    for i in range(NUM_HEADS):
        seed = jnp.where(h == i, jnp.float32(SEEDS_P1[i]), seed)

    pltpu.sync_copy(tok_hbm, tok_v)
    iota = lax.iota(jnp.int32, LANES)
    zeros = jnp.zeros((LANES,), jnp.float32)

    def zbody(j, c):
        for ci in range(N_SUB):
            acc[ci, pl.ds(j * LANES, LANES)] = zeros
        return c
    lax.fori_loop(0, POS // LANES, zbody, 0)

    def hash_stream(base_slot, n_entries, win_len, is_tri):
        magic = (1 << 21) // win_len + 1

        def body(i, c):
            e = i * LANES + iota
            valid = e < n_entries
            e = jnp.minimum(e, n_entries - 1)
            b = (e * magic) >> 21
            w = e - b * win_len
            p = b * T + w
            ti = b * TPAD + w
            t0 = plsc.load_gather(tok_v, [ti]).astype(jnp.float32)
            t1 = plsc.load_gather(tok_v, [ti + 1]).astype(jnp.float32)
            hv = t0 * seed + t1 * seed
            if is_tri:
                t2 = plsc.load_gather(tok_v, [ti + 2]).astype(jnp.float32)
                hv = hv + t2 * seed
            hidx = hv.astype(jnp.int32) & (HASH_RANGE - 1)
            hidx = jnp.where(valid, hidx, SENTINEL)
            idxl[pl.ds(base_slot + i * LANES, LANES)] = hidx
            posl[pl.ds(base_slot + i * LANES, LANES)] = p
            return c
        return body

    lax.fori_loop(0, NAV, hash_stream(0, NA, T - 1, False), 0, unroll=4)
    lax.fori_loop(0, NBV, hash_stream(NAV * LANES, NB, T - 2, True), 0,
                  unroll=4)

    nlv = LIST // LANES
    ci_vecs = [iota * 0 + ci for ci in range(N_SUB)]

    for s in range(N_SUB):
        pltpu.sync_copy(
            table_hbm.at[h, pl.ds(ca * N_SUB, N_SUB), pl.ds(s * SUB, SUB)],
            buf)

        def gbody(i, c):
            lv = idxl[pl.ds(i * LANES, LANES)]
            pv = posl[pl.ds(i * LANES, LANES)]
            m = (lv >> 13) == s
            rl = lv & (SUB - 1)
            for ci in range(N_SUB):
                val = plsc.load_gather(buf, [ci_vecs[ci], rl], mask=m)
                plsc.addupdate_scatter(acc, [ci_vecs[ci], pv], val, mask=m)
            return c
        lax.fori_loop(0, nlv, gbody, 0, unroll=8)

    pltpu.sync_copy(acc, out_hbm.at[pl.ds(wid * N_SUB, N_SUB)])


def _make_gather():
    mesh = plsc.VectorSubcoreMesh(
        core_axis_name="c", subcore_axis_name="s",
        num_cores=NC, num_subcores=NS)
    return pl.kernel(
        _gather_body,
        out_type=jax.ShapeDtypeStruct((NW * N_SUB, POS), jnp.float32),
        mesh=mesh,
        scratch_types=[
            pltpu.VMEM((B * TPAD,), jnp.int32),
            pltpu.VMEM((LIST,), jnp.int32),
            pltpu.VMEM((LIST,), jnp.int32),
            pltpu.VMEM((N_SUB, SUB), jnp.float32),
            pltpu.VMEM((N_SUB, POS), jnp.float32),
        ],
        compiler_params=pltpu.CompilerParams(
            use_tc_tiling_on_sc=True, needs_layout_passes=False),
    )


def _dense_body(parts_ref, hid_ref, wh_ref, bh_ref, wg1_ref, bg1_ref,
                wg2_ref, bg2_ref, out_ref):
    parts = parts_ref[...]
    seq_t = parts[0:64] + parts[64:128] + parts[128:192] + parts[192:256]
    seq = jnp.transpose(seq_t) * 0.25
    hid = hid_ref[...]
    dn = (((1,), (1,)), ((), ()))
    proj = lax.dot_general(seq, wh_ref[...], dn,
                           preferred_element_type=jnp.float32) + bh_ref[...]
    hmid = hid + proj
    g1 = lax.dot_general(hmid, wg1_ref[...], dn,
                         preferred_element_type=jnp.float32) + bg1_ref[...]
    g1 = 0.5 * g1 * (1.0 + lax.erf(g1 * jnp.float32(0.7071067811865476)))
    g2 = jnp.sum(g1 * wg2_ref[...], axis=1, keepdims=True) + bg2_ref[...]
    gate = jax.nn.sigmoid(g2)
    out_ref[...] = hid + gate * proj


def kernel(token_ids, hidden_state, embeddings, W_hid, b_hid,
           W_g1, b_g1, W_g2, b_g2):
    D = hidden_state.shape[-1]
    tok_pad = jnp.pad(token_ids, ((0, 0), (0, TPAD - T))).reshape(-1)
    table_t = embeddings.transpose(0, 2, 1)
    parts = _make_gather()(tok_pad, table_t)

    out = pl.pallas_call(
        _dense_body,
        out_shape=jax.ShapeDtypeStruct((POS, D), jnp.float32),
    )(
        parts,
        hidden_state.reshape(POS, D),
        W_hid,
        b_hid.reshape(1, D),
        W_g1,
        b_g1.reshape(1, -1),
        W_g2,
        b_g2.reshape(1, 1),
    )
    return out.reshape(B, T, D)


# speedup vs baseline: 1.3407x; 1.3034x over previous
import functools

import jax
import jax.numpy as jnp
from jax import lax
from jax.experimental import pallas as pl
from jax.experimental.pallas import tpu as pltpu
from jax.experimental.pallas import tpu_sc as plsc

SEEDS_P1 = (1609.0, 5154.0, 6527.0, 2426.0)
HASH_RANGE = 65536
NUM_HEADS = 4
EMBED_DIM = 64
LANES = 16
NC, NS = 2, 16
NW = NC * NS
N_SUB = 8
SUB = HASH_RANGE // N_SUB

B, T = 4, 512
TPAD = T + 8
POS = B * T
NA = B * (T - 1)
NB = B * (T - 2)
NAV = (NA + LANES - 1) // LANES
NBV = (NB + LANES - 1) // LANES
LIST = NAV * LANES + NBV * LANES
SENTINEL = HASH_RANGE


def _gather_body(tok_hbm, table_hbm, out_hbm,
                 tok_v, idxl, posl, lidx, lpos, buf, acc):
    wid = lax.axis_index("s") * NC + lax.axis_index("c")
    h = wid // N_SUB
    ca = wid - h * N_SUB
    seed = jnp.float32(0.0)
    for i in range(NUM_HEADS):
        seed = jnp.where(h == i, jnp.float32(SEEDS_P1[i]), seed)

    pltpu.sync_copy(tok_hbm, tok_v)
    iota = lax.iota(jnp.int32, LANES)
    zeros = jnp.zeros((LANES,), jnp.float32)

    def zbody(j, c):
        for ci in range(N_SUB):
            acc[ci, pl.ds(j * LANES, LANES)] = zeros
        return c
    lax.fori_loop(0, POS // LANES, zbody, 0)

    def hash_stream(base_slot, n_entries, win_len, is_tri):
        magic = (1 << 21) // win_len + 1

        def body(i, c):
            e = i * LANES + iota
            valid = e < n_entries
            e = jnp.minimum(e, n_entries - 1)
            b = (e * magic) >> 21
            w = e - b * win_len
            p = b * T + w
            ti = b * TPAD + w
            t0 = plsc.load_gather(tok_v, [ti]).astype(jnp.float32)
            t1 = plsc.load_gather(tok_v, [ti + 1]).astype(jnp.float32)
            hv = t0 * seed + t1 * seed
            if is_tri:
                t2 = plsc.load_gather(tok_v, [ti + 2]).astype(jnp.float32)
                hv = hv + t2 * seed
            hidx = hv.astype(jnp.int32) & (HASH_RANGE - 1)
            hidx = jnp.where(valid, hidx, SENTINEL)
            idxl[pl.ds(base_slot + i * LANES, LANES)] = hidx
            posl[pl.ds(base_slot + i * LANES, LANES)] = p
            return c
        return body

    lax.fori_loop(0, NAV, hash_stream(0, NA, T - 1, False), 0, unroll=4)
    lax.fori_loop(0, NBV, hash_stream(NAV * LANES, NB, T - 2, True), 0,
                  unroll=4)

    nlv = LIST // LANES
    ci_vecs = [iota * 0 + ci for ci in range(N_SUB)]

    for s in range(N_SUB):
        pltpu.sync_copy(
            table_hbm.at[h, pl.ds(ca * N_SUB, N_SUB), pl.ds(s * SUB, SUB)],
            buf)

        def cbody(i, off):
            lv = idxl[pl.ds(i * LANES, LANES)]
            pv = posl[pl.ds(i * LANES, LANES)]
            m = (lv >> 13) == s
            pf = plsc.cumsum(m.astype(jnp.int32))
            dest = off + pf - 1
            plsc.store_scatter(lidx, [dest], lv & (SUB - 1), mask=m)
            plsc.store_scatter(lpos, [dest], pv, mask=m)
            return off + pf[15]
        ns = lax.fori_loop(0, nlv, cbody, jnp.int32(0), unroll=4)

        def gbody(j, c):
            m = (j * LANES + iota) < ns
            rl = lidx[pl.ds(j * LANES, LANES)]
            pv = lpos[pl.ds(j * LANES, LANES)]
            for ci in range(N_SUB):
                val = plsc.load_gather(buf, [ci_vecs[ci], rl], mask=m)
                plsc.addupdate_scatter(acc, [ci_vecs[ci], pv], val, mask=m)
            return c
        nv = (ns + LANES - 1) >> 4
        lax.fori_loop(0, nv, gbody, 0)

    pltpu.sync_copy(acc, out_hbm.at[pl.ds(wid * N_SUB, N_SUB)])


def _make_gather():
    mesh = plsc.VectorSubcoreMesh(
        core_axis_name="c", subcore_axis_name="s",
        num_cores=NC, num_subcores=NS)
    return pl.kernel(
        _gather_body,
        out_type=jax.ShapeDtypeStruct((NW * N_SUB, POS), jnp.float32),
        mesh=mesh,
        scratch_types=[
            pltpu.VMEM((B * TPAD,), jnp.int32),
            pltpu.VMEM((LIST,), jnp.int32),
            pltpu.VMEM((LIST,), jnp.int32),
            pltpu.VMEM((LIST + LANES,), jnp.int32),
            pltpu.VMEM((LIST + LANES,), jnp.int32),
            pltpu.VMEM((N_SUB, SUB), jnp.float32),
            pltpu.VMEM((N_SUB, POS), jnp.float32),
        ],
        compiler_params=pltpu.CompilerParams(
            use_tc_tiling_on_sc=True, needs_layout_passes=False),
    )


def _dense_body(parts_ref, hid_ref, wh_ref, bh_ref, wg1_ref, bg1_ref,
                wg2_ref, bg2_ref, out_ref):
    parts = parts_ref[...]
    seq_t = parts[0:64] + parts[64:128] + parts[128:192] + parts[192:256]
    seq = jnp.transpose(seq_t) * 0.25
    hid = hid_ref[...]
    dn = (((1,), (1,)), ((), ()))
    proj = lax.dot_general(seq, wh_ref[...], dn,
                           preferred_element_type=jnp.float32) + bh_ref[...]
    hmid = hid + proj
    g1 = lax.dot_general(hmid, wg1_ref[...], dn,
                         preferred_element_type=jnp.float32) + bg1_ref[...]
    g1 = 0.5 * g1 * (1.0 + lax.erf(g1 * jnp.float32(0.7071067811865476)))
    g2 = jnp.sum(g1 * wg2_ref[...], axis=1, keepdims=True) + bg2_ref[...]
    gate = jax.nn.sigmoid(g2)
    out_ref[...] = hid + gate * proj


def kernel(token_ids, hidden_state, embeddings, W_hid, b_hid,
           W_g1, b_g1, W_g2, b_g2):
    D = hidden_state.shape[-1]
    tok_pad = jnp.pad(token_ids, ((0, 0), (0, TPAD - T))).reshape(-1)
    table_t = embeddings.transpose(0, 2, 1)
    parts = _make_gather()(tok_pad, table_t)

    out = pl.pallas_call(
        _dense_body,
        out_shape=jax.ShapeDtypeStruct((POS, D), jnp.float32),
    )(
        parts,
        hidden_state.reshape(POS, D),
        W_hid,
        b_hid.reshape(1, D),
        W_g1,
        b_g1.reshape(1, -1),
        W_g2,
        b_g2.reshape(1, 1),
    )
    return out.reshape(B, T, D)


# speedup vs baseline: 1.7405x; 1.2982x over previous
import functools

import jax
import jax.numpy as jnp
from jax import lax
from jax.experimental import pallas as pl
from jax.experimental.pallas import tpu as pltpu
from jax.experimental.pallas import tpu_sc as plsc

SEEDS_P1 = (1609.0, 5154.0, 6527.0, 2426.0)
HASH_RANGE = 65536
NUM_HEADS = 4
EMBED_DIM = 64
LANES = 16
NC, NS = 2, 16
NW = NC * NS
N_SUB = 8
SUB = HASH_RANGE // N_SUB

B, T = 4, 512
TPAD = T + 8
POS = B * T
NA = B * (T - 1)
NB = B * (T - 2)
NAV = (NA + LANES - 1) // LANES
NBV = (NB + LANES - 1) // LANES
LIST = NAV * LANES + NBV * LANES
SENTINEL = HASH_RANGE


def _gather_body(tok_hbm, table_hbm, out_hbm,
                 tok_v, idxl, posl, lidx, lpos, buf, acc, sem):
    wid = lax.axis_index("s") * NC + lax.axis_index("c")
    h = wid // N_SUB
    ca = wid - h * N_SUB
    seed = jnp.float32(0.0)
    for i in range(NUM_HEADS):
        seed = jnp.where(h == i, jnp.float32(SEEDS_P1[i]), seed)

    def slab_copy(s):
        return pltpu.async_copy(
            table_hbm.at[h, pl.ds(ca * N_SUB, N_SUB), pl.ds(s * SUB, SUB)],
            buf, sem)

    cp = slab_copy(0)
    pltpu.sync_copy(tok_hbm, tok_v)
    iota = lax.iota(jnp.int32, LANES)
    zeros = jnp.zeros((LANES,), jnp.float32)

    def zbody(j, c):
        for ci in range(N_SUB):
            acc[ci, pl.ds(j * LANES, LANES)] = zeros
        return c
    lax.fori_loop(0, POS // LANES, zbody, 0)

    def hash_stream(base_slot, n_entries, win_len, is_tri):
        magic = (1 << 21) // win_len + 1

        def body(i, c):
            e = i * LANES + iota
            valid = e < n_entries
            e = jnp.minimum(e, n_entries - 1)
            b = (e * magic) >> 21
            w = e - b * win_len
            p = b * T + w
            ti = b * TPAD + w
            t0 = plsc.load_gather(tok_v, [ti]).astype(jnp.float32)
            t1 = plsc.load_gather(tok_v, [ti + 1]).astype(jnp.float32)
            hv = t0 * seed + t1 * seed
            if is_tri:
                t2 = plsc.load_gather(tok_v, [ti + 2]).astype(jnp.float32)
                hv = hv + t2 * seed
            hidx = hv.astype(jnp.int32) & (HASH_RANGE - 1)
            hidx = jnp.where(valid, hidx, SENTINEL)
            idxl[pl.ds(base_slot + i * LANES, LANES)] = hidx
            posl[pl.ds(base_slot + i * LANES, LANES)] = p
            return c
        return body

    lax.fori_loop(0, NAV, hash_stream(0, NA, T - 1, False), 0, unroll=4)
    lax.fori_loop(0, NBV, hash_stream(NAV * LANES, NB, T - 2, True), 0,
                  unroll=4)

    nlv = LIST // LANES
    ci_vecs = [iota * 0 + ci for ci in range(N_SUB)]

    for s in range(N_SUB):
        def cbody(i, off):
            lv = idxl[pl.ds(i * LANES, LANES)]
            pv = posl[pl.ds(i * LANES, LANES)]
            m = (lv >> 13) == s
            pf = plsc.cumsum(m.astype(jnp.int32))
            dest = off + pf - 1
            plsc.store_scatter(lidx, [dest], lv & (SUB - 1), mask=m)
            plsc.store_scatter(lpos, [dest], pv, mask=m)
            return off + pf[15]
        ns = lax.fori_loop(0, nlv, cbody, jnp.int32(0), unroll=4)
        cp.wait()

        def gbody(j, c):
            m = (j * LANES + iota) < ns
            rl = lidx[pl.ds(j * LANES, LANES)]
            pv = lpos[pl.ds(j * LANES, LANES)]
            for ci in range(N_SUB):
                val = plsc.load_gather(buf, [ci_vecs[ci], rl], mask=m)
                plsc.addupdate_scatter(acc, [ci_vecs[ci], pv], val, mask=m)
            return c
        nv = (ns + LANES - 1) >> 4
        lax.fori_loop(0, nv, gbody, 0)
        if s + 1 < N_SUB:
            cp = slab_copy(s + 1)

    pltpu.sync_copy(acc, out_hbm.at[pl.ds(wid * N_SUB, N_SUB)])


def _make_gather():
    mesh = plsc.VectorSubcoreMesh(
        core_axis_name="c", subcore_axis_name="s",
        num_cores=NC, num_subcores=NS)
    return pl.kernel(
        _gather_body,
        out_type=jax.ShapeDtypeStruct((NW * N_SUB, POS), jnp.float32),
        mesh=mesh,
        scratch_types=[
            pltpu.VMEM((B * TPAD,), jnp.int32),
            pltpu.VMEM((LIST,), jnp.int32),
            pltpu.VMEM((LIST,), jnp.int32),
            pltpu.VMEM((LIST + LANES,), jnp.int32),
            pltpu.VMEM((LIST + LANES,), jnp.int32),
            pltpu.VMEM((N_SUB, SUB), jnp.float32),
            pltpu.VMEM((N_SUB, POS), jnp.float32),
            pltpu.SemaphoreType.DMA,
        ],
        compiler_params=pltpu.CompilerParams(
            use_tc_tiling_on_sc=True, needs_layout_passes=False),
    )


def _dense_body(parts_ref, hid_ref, wh_ref, bh_ref, wg1_ref, bg1_ref,
                wg2_ref, bg2_ref, out_ref):
    parts = parts_ref[...]
    seq_t = parts[0:64] + parts[64:128] + parts[128:192] + parts[192:256]
    seq = jnp.transpose(seq_t) * 0.25
    hid = hid_ref[...]
    dn = (((1,), (1,)), ((), ()))
    proj = lax.dot_general(seq, wh_ref[...], dn,
                           preferred_element_type=jnp.float32) + bh_ref[...]
    hmid = hid + proj
    g1 = lax.dot_general(hmid, wg1_ref[...], dn,
                         preferred_element_type=jnp.float32) + bg1_ref[...]
    g1 = 0.5 * g1 * (1.0 + lax.erf(g1 * jnp.float32(0.7071067811865476)))
    g2 = jnp.sum(g1 * wg2_ref[...], axis=1, keepdims=True) + bg2_ref[...]
    gate = jax.nn.sigmoid(g2)
    out_ref[...] = hid + gate * proj


def kernel(token_ids, hidden_state, embeddings, W_hid, b_hid,
           W_g1, b_g1, W_g2, b_g2):
    D = hidden_state.shape[-1]
    tok_pad = jnp.pad(token_ids, ((0, 0), (0, TPAD - T))).reshape(-1)
    table_t = embeddings.transpose(0, 2, 1)
    parts = _make_gather()(tok_pad, table_t)

    out = pl.pallas_call(
        _dense_body,
        out_shape=jax.ShapeDtypeStruct((POS, D), jnp.float32),
    )(
        parts,
        hidden_state.reshape(POS, D),
        W_hid,
        b_hid.reshape(1, D),
        W_g1,
        b_g1.reshape(1, -1),
        W_g2,
        b_g2.reshape(1, 1),
    )
    return out.reshape(B, T, D)
